# Initial kernel scaffold; baseline (speedup 1.0000x reference)
import jax
import jax.numpy as jnp
from jax.experimental import pallas as pl


def kernel(pcdag, W1, b1, W2, b2, Wfc, bfc):
    raise NotImplementedError("write your pallas kernel here")


# speedup vs baseline: 4130.9852x; 4130.9852x over previous
import jax
import jax.numpy as jnp
from jax.experimental import pallas as pl

_N = 2048
_BC = 512
_BF = 512


def _prep_body(a_ref, abf_ref, dinv_ref):
    a = a_ref[...]
    abf_ref[...] = a.astype(jnp.bfloat16)
    cs = jnp.sum(a, axis=0, keepdims=True)
    d = 1.0 / jnp.sqrt(cs + 1.0)
    dinv_ref[...] = jnp.broadcast_to(d, (8, _BC))


def _layer1_body(abf_ref, dinv_ref, dinvb_ref, w1t_ref, w1tb_ref, b1_ref,
                 x1_ref):
    dinv = dinv_ref[0:1, :]
    t1 = dinv * w1t_ref[...]
    a = abf_ref[...].astype(jnp.float32)
    h = jax.lax.dot_general(t1, a, (((1,), (0,)), ((), ())),
                            preferred_element_type=jnp.float32)
    db = dinvb_ref[0:1, :]
    t1b = db * w1tb_ref[...]
    x1_ref[...] = jnp.maximum(db * (h + t1b) + b1_ref[:, 0:1], 0.0)


def _layer2_body(abf_ref, dinv_ref, dinvb_ref, x1_ref, x1b_ref, w2t_ref,
                 b2_ref, wcols_ref, wrows_ref, bfcb_ref, acol_ref, bcol_ref,
                 arow_ref, brow_ref):
    dinv = dinv_ref[0:1, :]
    xw = jax.lax.dot_general(w2t_ref[...], x1_ref[...], (((1,), (0,)), ((), ())),
                             preferred_element_type=jnp.float32)
    t2 = dinv * xw
    a = abf_ref[...].astype(jnp.float32)
    h = jax.lax.dot_general(t2, a, (((1,), (0,)), ((), ())),
                            preferred_element_type=jnp.float32)
    db = dinvb_ref[0:1, :]
    t2b = db * jax.lax.dot_general(w2t_ref[...], x1b_ref[...],
                                   (((1,), (0,)), ((), ())),
                                   preferred_element_type=jnp.float32)
    x2 = jnp.maximum(db * (h + t2b) + b2_ref[:, 0:1], 0.0)
    bfc = bfcb_ref[0:1, 0:1]
    rows = jax.lax.dot_general(wrows_ref[...], x2, (((1,), (0,)), ((), ())),
                               preferred_element_type=jnp.float32)
    cols = jax.lax.dot_general(x2, wcols_ref[...], (((0,), (0,)), ((), ())),
                               preferred_element_type=jnp.float32)
    arow_ref[...] = rows[0:8, :] + bfc
    brow_ref[...] = rows[8:16, :]
    acol_ref[...] = cols[:, 0:8] + bfc
    bcol_ref[...] = cols[:, 8:16]


def _orient_body(ab_ref, at_ref, acol_ref, bcol_ref, arow_ref, brow_ref,
                 out_ref):
    i = pl.program_id(0)
    j = pl.program_id(1)
    ab = ab_ref[...].astype(jnp.float32)
    at = at_ref[...].astype(jnp.float32)
    mask = (ab == 1.0) & (jnp.transpose(at) == 1.0)
    e = acol_ref[...][:, 0:1] + brow_ref[...][0:1, :]
    et = bcol_ref[...][:, 0:1] + arow_ref[...][0:1, :]
    rid = i * _BF + jax.lax.broadcasted_iota(jnp.int32, (_BF, _BF), 0)
    cid = j * _BF + jax.lax.broadcasted_iota(jnp.int32, (_BF, _BF), 1)
    upper = rid < cid
    val = jax.nn.sigmoid(jnp.where(upper, e, -et))
    out_ref[...] = jnp.where(mask, val, ab)


def kernel(pcdag, W1, b1, W2, b2, Wfc, bfc):
    n = _N
    a = pcdag.astype(jnp.float32)
    w1t = W1.T
    w2t = W2.T
    b1c = jnp.broadcast_to(b1.reshape(16, 1), (16, 8))
    b2c = jnp.broadcast_to(b2.reshape(8, 1), (8, 8))
    top = Wfc[0:8, 0]
    bot = Wfc[8:16, 0]
    wcols = jnp.concatenate([jnp.broadcast_to(top[:, None], (8, 8)),
                             jnp.broadcast_to(bot[:, None], (8, 8))], axis=1)
    wrows = jnp.concatenate([jnp.broadcast_to(top[None, :], (8, 8)),
                             jnp.broadcast_to(bot[None, :], (8, 8))], axis=0)
    bfcb = jnp.broadcast_to(bfc.reshape(1, 1), (8, 8))

    nblk = n // _BC
    abf, dinv = pl.pallas_call(
        _prep_body,
        grid=(nblk,),
        in_specs=[pl.BlockSpec((n, _BC), lambda j: (0, j))],
        out_specs=[pl.BlockSpec((n, _BC), lambda j: (0, j)),
                   pl.BlockSpec((8, _BC), lambda j: (0, j))],
        out_shape=[jax.ShapeDtypeStruct((n, n), jnp.bfloat16),
                   jax.ShapeDtypeStruct((8, n), jnp.float32)],
    )(a)

    x1 = pl.pallas_call(
        _layer1_body,
        grid=(nblk,),
        in_specs=[pl.BlockSpec((n, _BC), lambda j: (0, j)),
                  pl.BlockSpec((8, n), lambda j: (0, 0)),
                  pl.BlockSpec((8, _BC), lambda j: (0, j)),
                  pl.BlockSpec((16, n), lambda j: (0, 0)),
                  pl.BlockSpec((16, _BC), lambda j: (0, j)),
                  pl.BlockSpec((16, 8), lambda j: (0, 0))],
        out_specs=pl.BlockSpec((16, _BC), lambda j: (0, j)),
        out_shape=jax.ShapeDtypeStruct((16, n), jnp.float32),
    )(abf, dinv, dinv, w1t, w1t, b1c)

    acol, bcol, arow, brow = pl.pallas_call(
        _layer2_body,
        grid=(nblk,),
        in_specs=[pl.BlockSpec((n, _BC), lambda j: (0, j)),
                  pl.BlockSpec((8, n), lambda j: (0, 0)),
                  pl.BlockSpec((8, _BC), lambda j: (0, j)),
                  pl.BlockSpec((16, n), lambda j: (0, 0)),
                  pl.BlockSpec((16, _BC), lambda j: (0, j)),
                  pl.BlockSpec((8, 16), lambda j: (0, 0)),
                  pl.BlockSpec((8, 8), lambda j: (0, 0)),
                  pl.BlockSpec((8, 16), lambda j: (0, 0)),
                  pl.BlockSpec((16, 8), lambda j: (0, 0)),
                  pl.BlockSpec((8, 8), lambda j: (0, 0))],
        out_specs=[pl.BlockSpec((_BC, 8), lambda j: (j, 0)),
                   pl.BlockSpec((_BC, 8), lambda j: (j, 0)),
                   pl.BlockSpec((8, _BC), lambda j: (0, j)),
                   pl.BlockSpec((8, _BC), lambda j: (0, j))],
        out_shape=[jax.ShapeDtypeStruct((n, 8), jnp.float32),
                   jax.ShapeDtypeStruct((n, 8), jnp.float32),
                   jax.ShapeDtypeStruct((8, n), jnp.float32),
                   jax.ShapeDtypeStruct((8, n), jnp.float32)],
    )(abf, dinv, dinv, x1, x1, w2t, b2c, wcols, wrows, bfcb)

    nf = n // _BF
    out = pl.pallas_call(
        _orient_body,
        grid=(nf, nf),
        in_specs=[pl.BlockSpec((_BF, _BF), lambda i, j: (i, j)),
                  pl.BlockSpec((_BF, _BF), lambda i, j: (j, i)),
                  pl.BlockSpec((_BF, 8), lambda i, j: (i, 0)),
                  pl.BlockSpec((_BF, 8), lambda i, j: (i, 0)),
                  pl.BlockSpec((8, _BF), lambda i, j: (0, j)),
                  pl.BlockSpec((8, _BF), lambda i, j: (0, j))],
        out_specs=pl.BlockSpec((_BF, _BF), lambda i, j: (i, j)),
        out_shape=jax.ShapeDtypeStruct((n, n), jnp.float32),
    )(abf, abf, acol, bcol, arow, brow)
    return out
